# baseline (speedup 1.0000x reference)
import jax
import jax.numpy as jnp
from jax import lax
from jax.experimental import pallas as pl
from jax.experimental.pallas import tpu as pltpu

_CH = 64


def _fused(table, idx0, idx1, W, b2d, seq_len):
    B = idx0.shape[0]
    D = table.shape[1]
    nch = B // _CH

    def body(i0_ref, i1_ref, table_ref, w_hbm, b_ref, o_ref,
             x_v, w_v, wsem, csems):
        pltpu.make_async_copy(w_hbm, w_v, wsem).start()

        def issue_row(r, _):
            flat = i0_ref[r] * seq_len + i1_ref[r]
            pltpu.make_async_copy(
                table_ref.at[flat], x_v.at[r], csems.at[r // _CH]
            ).start()
            return _

        lax.fori_loop(0, B, issue_row, 0, unroll=8)

        pltpu.make_async_copy(w_hbm, w_v, wsem).wait()
        for c in range(nch):
            sl = pl.ds(c * _CH, _CH)
            pltpu.make_async_copy(
                table_ref.at[pl.ds(0, _CH)], x_v.at[sl], csems.at[c]
            ).wait()
            acc = lax.dot_general(
                x_v[sl, :].astype(jnp.bfloat16), w_v[...],
                (((1,), (1,)), ((), ())),
                preferred_element_type=jnp.float32,
            )
            o_ref[sl, :] = jnp.tanh(acc + b_ref[...])

    return pl.pallas_call(
        body,
        in_specs=[
            pl.BlockSpec(memory_space=pltpu.SMEM),
            pl.BlockSpec(memory_space=pltpu.SMEM),
            pl.BlockSpec(memory_space=pltpu.HBM),
            pl.BlockSpec(memory_space=pltpu.HBM),
            pl.BlockSpec(memory_space=pltpu.VMEM),
        ],
        out_specs=pl.BlockSpec(memory_space=pltpu.VMEM),
        out_shape=jax.ShapeDtypeStruct((B, D), jnp.float32),
        scratch_shapes=[
            pltpu.VMEM((B, D), jnp.float32),
            pltpu.VMEM((D, D), jnp.bfloat16),
            pltpu.SemaphoreType.DMA,
            pltpu.SemaphoreType.DMA((nch,)),
        ],
    )(idx0, idx1, table, W, b2d)


def kernel(hidden_states, cls_indexes, W, b):
    n_batch, seq_len, D = hidden_states.shape
    table = hidden_states.reshape(n_batch * seq_len, D)
    idx = cls_indexes.astype(jnp.int32)
    return _fused(table, idx[:, 0], idx[:, 1],
                  W.astype(jnp.bfloat16), b.reshape(1, D), seq_len)


# speedup vs baseline: 1.6961x; 1.6961x over previous
import jax
import jax.numpy as jnp
from jax import lax
from jax.experimental import pallas as pl
from jax.experimental.pallas import tpu as pltpu

_CH = 512


def _fused(table, idx0, idx1, W, b2d, seq_len):
    B = idx0.shape[0]
    D = table.shape[1]
    nch = B // _CH

    def body(i0_ref, i1_ref, table_ref, w_hbm, b_ref, o_ref,
             x_v, w_v, wsem, csems):
        pltpu.make_async_copy(w_hbm, w_v, wsem).start()

        def issue_row(r, _):
            flat = i0_ref[r] * seq_len + i1_ref[r]
            pltpu.make_async_copy(
                table_ref.at[flat], x_v.at[r], csems.at[r // _CH]
            ).start()
            return _

        lax.fori_loop(0, B, issue_row, 0, unroll=8)

        pltpu.make_async_copy(w_hbm, w_v, wsem).wait()
        for c in range(nch):
            sl = pl.ds(c * _CH, _CH)
            pltpu.make_async_copy(
                table_ref.at[pl.ds(0, _CH)], x_v.at[sl], csems.at[c]
            ).wait()
            acc = lax.dot_general(
                x_v[sl, :], w_v[...],
                (((1,), (1,)), ((), ())),
                preferred_element_type=jnp.float32,
            )
            o_ref[sl, :] = jnp.tanh(acc + b_ref[...])

    return pl.pallas_call(
        body,
        in_specs=[
            pl.BlockSpec(memory_space=pltpu.SMEM),
            pl.BlockSpec(memory_space=pltpu.SMEM),
            pl.BlockSpec(memory_space=pltpu.HBM),
            pl.BlockSpec(memory_space=pltpu.HBM),
            pl.BlockSpec(memory_space=pltpu.VMEM),
        ],
        out_specs=pl.BlockSpec(memory_space=pltpu.VMEM),
        out_shape=jax.ShapeDtypeStruct((B, D), jnp.float32),
        scratch_shapes=[
            pltpu.VMEM((B, D), jnp.float32),
            pltpu.VMEM((D, D), jnp.float32),
            pltpu.SemaphoreType.DMA,
            pltpu.SemaphoreType.DMA((nch,)),
        ],
    )(idx0, idx1, table, W, b2d)


def kernel(hidden_states, cls_indexes, W, b):
    n_batch, seq_len, D = hidden_states.shape
    table = hidden_states.reshape(n_batch * seq_len, D)
    idx = cls_indexes.astype(jnp.int32)
    return _fused(table, idx[:, 0], idx[:, 1], W, b.reshape(1, D), seq_len)


# speedup vs baseline: 1.7805x; 1.0498x over previous
import jax
import jax.numpy as jnp
from jax import lax
from jax.experimental import pallas as pl
from jax.experimental.pallas import tpu as pltpu

_CH = 256


def _fused(table, idx0, idx1, W, b2d, seq_len):
    B = idx0.shape[0]
    D = table.shape[1]
    nch = B // _CH

    def body(i0_ref, i1_ref, table_ref, w_hbm, b_ref, o_ref,
             x_v, w_v, wsem, csems):
        pltpu.make_async_copy(w_hbm, w_v, wsem).start()

        def issue_row(r, _):
            flat = i0_ref[r] * seq_len + i1_ref[r]
            pltpu.make_async_copy(
                table_ref.at[flat], x_v.at[r], csems.at[r // _CH]
            ).start()
            return _

        lax.fori_loop(0, B, issue_row, 0, unroll=8)

        pltpu.make_async_copy(w_hbm, w_v, wsem).wait()
        for c in range(nch):
            sl = pl.ds(c * _CH, _CH)
            pltpu.make_async_copy(
                table_ref.at[pl.ds(0, _CH)], x_v.at[sl], csems.at[c]
            ).wait()
            acc = lax.dot_general(
                x_v[sl, :], w_v[...],
                (((1,), (1,)), ((), ())),
                preferred_element_type=jnp.float32,
            )
            o_ref[sl, :] = jnp.tanh(acc + b_ref[...])

    return pl.pallas_call(
        body,
        in_specs=[
            pl.BlockSpec(memory_space=pltpu.SMEM),
            pl.BlockSpec(memory_space=pltpu.SMEM),
            pl.BlockSpec(memory_space=pltpu.HBM),
            pl.BlockSpec(memory_space=pltpu.HBM),
            pl.BlockSpec(memory_space=pltpu.VMEM),
        ],
        out_specs=pl.BlockSpec(memory_space=pltpu.VMEM),
        out_shape=jax.ShapeDtypeStruct((B, D), jnp.float32),
        scratch_shapes=[
            pltpu.VMEM((B, D), jnp.float32),
            pltpu.VMEM((D, D), jnp.float32),
            pltpu.SemaphoreType.DMA,
            pltpu.SemaphoreType.DMA((nch,)),
        ],
    )(idx0, idx1, table, W, b2d)


def kernel(hidden_states, cls_indexes, W, b):
    n_batch, seq_len, D = hidden_states.shape
    table = hidden_states.reshape(n_batch * seq_len, D)
    idx = cls_indexes.astype(jnp.int32)
    return _fused(table, idx[:, 0], idx[:, 1], W, b.reshape(1, D), seq_len)
